# baseline (speedup 1.0000x reference)
import dataclasses
import functools

import jax
import jax.numpy as jnp
from jax import lax
from jax.experimental import pallas as pl
from jax.experimental.pallas import tpu as pltpu
from jax.experimental.pallas import tpu_sc as plsc

_RATE = 0.1

_L = 16
_NC = 2
_NS = 16
_N = 1_000_000
_NT = 128
_TPW = 8


def _copy_body(x_ref, o_ref, sem):
    pltpu.make_async_copy(x_ref, o_ref, sem).start()
    pltpu.make_async_copy(x_ref, o_ref, sem).wait()


_tc_copy = pl.pallas_call(
    _copy_body,
    in_specs=[pl.BlockSpec(memory_space=pltpu.MemorySpace.HBM)],
    out_specs=pl.BlockSpec(memory_space=pltpu.MemorySpace.HBM),
    out_shape=jax.ShapeDtypeStruct((_N,), jnp.float32),
    scratch_shapes=[pltpu.SemaphoreType.DMA],
)


@functools.lru_cache(maxsize=1)
def _build_fixup_kernel():
    mesh = plsc.VectorSubcoreMesh(
        core_axis_name="c", subcore_axis_name="s",
        num_cores=_NC, num_subcores=_NS,
    )
    cp = pltpu.CompilerParams()
    if "needs_layout_passes" in pltpu.CompilerParams.__dataclass_fields__:
        cp = dataclasses.replace(cp, needs_layout_passes=False)
    if "use_tc_tiling_on_sc" in pltpu.CompilerParams.__dataclass_fields__:
        cp = dataclasses.replace(cp, use_tc_tiling_on_sc=False)
    if "skip_device_barrier" in pltpu.CompilerParams.__dataclass_fields__:
        cp = dataclasses.replace(cp, skip_device_barrier=True)

    @functools.partial(
        pl.kernel,
        compiler_params=cp,
        out_type=(),
        mesh=mesh,
        scratch_types=[
            pltpu.VMEM((_TPW, _L), jnp.float32),
            pltpu.VMEM((_NT,), jnp.int32),
            pltpu.VMEM((_NT,), jnp.float32),
            pltpu.VMEM((_L,), jnp.float32),
            [pltpu.SemaphoreType.DMA] * _TPW,
        ],
    )
    def _fixup(y_hbm, tgt_hbm, stim_hbm, tmod_hbm,
               wbuf, tgt_v, stim_v, tmod_v, sems):
        c = lax.axis_index("c")
        s = lax.axis_index("s")
        wid = s * _NC + c

        @pl.when(wid < _NT // _TPW)
        def _active():
            pltpu.sync_copy(tgt_hbm, tgt_v)
            pltpu.sync_copy(stim_hbm, stim_v)
            pltpu.sync_copy(tmod_hbm, tmod_v)
            gate = jnp.where(
                tmod_v[...] == 0.0,
                jnp.full((_L,), 1.0, jnp.float32),
                jnp.full((_L,), 0.0, jnp.float32),
            )
            lane = lax.iota(jnp.int32, _L)
            grp = (wid // 2) * _L
            half = (wid % 2) * _TPW
            tvec = tgt_v[pl.ds(grp, _L)]
            svec = stim_v[pl.ds(grp, _L)] * gate
            offs = []
            cols = []
            vals = []
            ins = []
            for i in range(_TPW):
                ln = half + i
                sel = lane == ln
                t_i = lax.reduce_max(jnp.where(sel, tvec, 0), axes=(0,))
                s_i = lax.reduce_sum(
                    jnp.where(sel, svec, jnp.zeros_like(svec)), axes=(0,))
                col = lax.rem(t_i, _L)
                off = pl.multiple_of(t_i - col, _L)
                offs.append(off)
                cols.append(col)
                vals.append(s_i)
                ins.append(pltpu.async_copy(
                    y_hbm.at[pl.ds(off, _L)], wbuf.at[i], sems[i]))
            for i in range(_TPW):
                ins[i].wait()
                add = jnp.where(lane == cols[i],
                                jnp.full((_L,), 1.0, jnp.float32),
                                jnp.full((_L,), 0.0, jnp.float32)) * vals[i]
                wbuf[i] = wbuf[i] + add
            outs = [
                pltpu.async_copy(
                    wbuf.at[i], y_hbm.at[pl.ds(offs[i], _L)], sems[i])
                for i in range(_TPW)
            ]
            for o in outs:
                o.wait()

    return _fixup


def kernel(t, out, targets, stimulation_strength):
    tmod = (t % (1.0 / _RATE)).astype(jnp.float32)
    tmod_vec = jnp.broadcast_to(tmod, (_L,))
    tgt = targets.astype(jnp.int32)
    y = _tc_copy(out)
    yref = jax.new_ref(y)
    _build_fixup_kernel()(yref, tgt, stimulation_strength, tmod_vec)
    return yref[...]


# speedup vs baseline: 5.2445x; 5.2445x over previous
import dataclasses
import functools

import jax
import jax.numpy as jnp
from jax import lax
from jax.experimental import pallas as pl
from jax.experimental.pallas import tpu as pltpu
from jax.experimental.pallas import tpu_sc as plsc

_RATE = 0.1

_L = 16
_NC = 2
_NS = 16
_N = 1_000_000
_NT = 128
_TPW = 8


_BLK = 131072
_NBLK = -(-_N // _BLK)


def _copy_body(x_ref, o_ref):
    o_ref[...] = x_ref[...]


_tc_copy = pl.pallas_call(
    _copy_body,
    grid=(_NBLK,),
    in_specs=[pl.BlockSpec((_BLK,), lambda i: (i,))],
    out_specs=pl.BlockSpec((_BLK,), lambda i: (i,)),
    out_shape=jax.ShapeDtypeStruct((_N,), jnp.float32),
)


@functools.lru_cache(maxsize=1)
def _build_fixup_kernel():
    mesh = plsc.VectorSubcoreMesh(
        core_axis_name="c", subcore_axis_name="s",
        num_cores=_NC, num_subcores=_NS,
    )
    cp = pltpu.CompilerParams()
    if "needs_layout_passes" in pltpu.CompilerParams.__dataclass_fields__:
        cp = dataclasses.replace(cp, needs_layout_passes=False)
    if "use_tc_tiling_on_sc" in pltpu.CompilerParams.__dataclass_fields__:
        cp = dataclasses.replace(cp, use_tc_tiling_on_sc=False)
    if "skip_device_barrier" in pltpu.CompilerParams.__dataclass_fields__:
        cp = dataclasses.replace(cp, skip_device_barrier=True)

    @functools.partial(
        pl.kernel,
        compiler_params=cp,
        out_type=(),
        mesh=mesh,
        scratch_types=[
            pltpu.VMEM((_TPW, _L), jnp.float32),
            pltpu.VMEM((_NT,), jnp.int32),
            pltpu.VMEM((_NT,), jnp.float32),
            pltpu.VMEM((_L,), jnp.float32),
            [pltpu.SemaphoreType.DMA] * _TPW,
        ],
    )
    def _fixup(y_hbm, tgt_hbm, stim_hbm, tmod_hbm,
               wbuf, tgt_v, stim_v, tmod_v, sems):
        c = lax.axis_index("c")
        s = lax.axis_index("s")
        wid = s * _NC + c

        @pl.when(wid < _NT // _TPW)
        def _active():
            pltpu.sync_copy(tgt_hbm, tgt_v)
            pltpu.sync_copy(stim_hbm, stim_v)
            pltpu.sync_copy(tmod_hbm, tmod_v)
            gate = jnp.where(
                tmod_v[...] == 0.0,
                jnp.full((_L,), 1.0, jnp.float32),
                jnp.full((_L,), 0.0, jnp.float32),
            )
            lane = lax.iota(jnp.int32, _L)
            grp = (wid // 2) * _L
            half = (wid % 2) * _TPW
            tvec = tgt_v[pl.ds(grp, _L)]
            svec = stim_v[pl.ds(grp, _L)] * gate
            offs = []
            cols = []
            vals = []
            ins = []
            for i in range(_TPW):
                ln = half + i
                sel = lane == ln
                t_i = lax.reduce_max(jnp.where(sel, tvec, 0), axes=(0,))
                s_i = lax.reduce_sum(
                    jnp.where(sel, svec, jnp.zeros_like(svec)), axes=(0,))
                col = lax.rem(t_i, _L)
                off = pl.multiple_of(t_i - col, _L)
                offs.append(off)
                cols.append(col)
                vals.append(s_i)
                ins.append(pltpu.async_copy(
                    y_hbm.at[pl.ds(off, _L)], wbuf.at[i], sems[i]))
            for i in range(_TPW):
                ins[i].wait()
                add = jnp.where(lane == cols[i],
                                jnp.full((_L,), 1.0, jnp.float32),
                                jnp.full((_L,), 0.0, jnp.float32)) * vals[i]
                wbuf[i] = wbuf[i] + add
            outs = [
                pltpu.async_copy(
                    wbuf.at[i], y_hbm.at[pl.ds(offs[i], _L)], sems[i])
                for i in range(_TPW)
            ]
            for o in outs:
                o.wait()

    return _fixup


def kernel(t, out, targets, stimulation_strength):
    tmod = (t % (1.0 / _RATE)).astype(jnp.float32)
    tmod_vec = jnp.broadcast_to(tmod, (_L,))
    tgt = targets.astype(jnp.int32)
    y = _tc_copy(out)
    yref = jax.new_ref(y)
    _build_fixup_kernel()(yref, tgt, stimulation_strength, tmod_vec)
    return yref[...]


# speedup vs baseline: 5.6875x; 1.0845x over previous
import dataclasses
import functools

import jax
import jax.numpy as jnp
from jax import lax
from jax.experimental import pallas as pl
from jax.experimental.pallas import tpu as pltpu
from jax.experimental.pallas import tpu_sc as plsc

_RATE = 0.1

_L = 16
_NC = 2
_NS = 16
_N = 1_000_000
_NT = 128

_HE = 2192 * _L
_LE = 1712 * _L
_TAIL = _N - _NS * (_HE + _LE)
_LMAX = _LE + _TAIL
_HEAVY_CORE = 1


def _apply_adds(buf, tgt_v, stim_v, gate_zero, base, nelems):
    for j in range(_NT // _L):
        t = tgt_v[pl.ds(j * _L, _L)]
        s = stim_v[pl.ds(j * _L, _L)]
        s = jnp.where(gate_zero, s, jnp.zeros_like(s))
        et = t - base
        inb = (et >= 0) & (et < nelems)
        et_c = jnp.minimum(jnp.maximum(et, 0), nelems - 1)
        plsc.addupdate_scatter(buf, [et_c], s, mask=inb)


@functools.lru_cache(maxsize=1)
def _build_stim_kernel():
    mesh = plsc.VectorSubcoreMesh(
        core_axis_name="c", subcore_axis_name="s",
        num_cores=_NC, num_subcores=_NS,
    )
    cp = pltpu.CompilerParams()
    if "needs_layout_passes" in pltpu.CompilerParams.__dataclass_fields__:
        cp = dataclasses.replace(cp, needs_layout_passes=False)
    if "use_tc_tiling_on_sc" in pltpu.CompilerParams.__dataclass_fields__:
        cp = dataclasses.replace(cp, use_tc_tiling_on_sc=False)
    if "skip_device_barrier" in pltpu.CompilerParams.__dataclass_fields__:
        cp = dataclasses.replace(cp, skip_device_barrier=True)

    @functools.partial(
        pl.kernel,
        compiler_params=cp,
        out_type=jax.ShapeDtypeStruct((_N,), jnp.float32),
        mesh=mesh,
        scratch_types=[
            pltpu.VMEM((_LMAX if _LMAX > _HE else _HE,), jnp.float32),
            pltpu.VMEM((_NT,), jnp.int32),
            pltpu.VMEM((_NT,), jnp.float32),
            pltpu.VMEM((_L,), jnp.float32),
            pltpu.SemaphoreType.DMA,
        ],
    )
    def _stim_kernel(x_hbm, tgt_hbm, stim_hbm, tmod_hbm, o_hbm,
                     buf, tgt_v, stim_v, tmod_v, sem):
        c = lax.axis_index("c")
        s = lax.axis_index("s")

        def do_chunk(base, nelems):
            bslc = buf.at[pl.ds(0, nelems)]
            cin = pltpu.async_copy(x_hbm.at[pl.ds(base, nelems)], bslc, sem)
            pltpu.sync_copy(tgt_hbm, tgt_v)
            pltpu.sync_copy(stim_hbm, stim_v)
            pltpu.sync_copy(tmod_hbm, tmod_v)
            gate_zero = tmod_v[...] == 0.0
            cin.wait()
            _apply_adds(bslc, tgt_v, stim_v, gate_zero, base, nelems)
            pltpu.async_copy(bslc, o_hbm.at[pl.ds(base, nelems)], sem).wait()

        @pl.when(c == _HEAVY_CORE)
        def _heavy():
            do_chunk(s * _HE, _HE)

        @pl.when((c != _HEAVY_CORE) & (s < _NS - 1))
        def _light():
            do_chunk(_NS * _HE + s * _LE, _LE)

        @pl.when((c != _HEAVY_CORE) & (s == _NS - 1))
        def _light_tail():
            do_chunk(_NS * _HE + (_NS - 1) * _LE, _LMAX)

    return _stim_kernel


def kernel(t, out, targets, stimulation_strength):
    tmod = (t % (1.0 / _RATE)).astype(jnp.float32)
    tmod_vec = jnp.broadcast_to(tmod, (_L,))
    tgt = targets.astype(jnp.int32)
    return _build_stim_kernel()(out, tgt, stimulation_strength, tmod_vec)
